# baseline (device time: 56717 ns/iter reference)
import jax
import jax.numpy as jnp
from jax import lax
from jax.experimental import pallas as pl
from jax.experimental.pallas import tpu as pltpu

K = 16
N_PARTS = 8
IDX_MASK = 0xFFF
KEY_MASK = ~0xFFF
NEG = -(2 ** 31)
SIGN_FIX = 0x7FFFFFFF


def _to_keys(vals_f32, col_iota):
    bits = lax.bitcast_convert_type(vals_f32, jnp.int32)
    mkey = jnp.where(bits >= 0, bits, bits ^ SIGN_FIX)
    return (mkey & KEY_MASK) | col_iota


def _from_key(key):
    mk = key & KEY_MASK
    bits = jnp.where(mk >= 0, mk, mk ^ SIGN_FIX)
    return lax.bitcast_convert_type(bits, jnp.float32)


def _successive_maxima(keys, reduce_axes, rows):
    out_pos = lax.broadcasted_iota(jnp.int32, (rows, K), 1)
    mx = jnp.max(keys, axis=reduce_axes)
    acc = jnp.broadcast_to(mx[:, None], (rows, K))
    for j in range(1, K):
        masked = jnp.where(keys < jnp.expand_dims(mx, reduce_axes), keys, NEG)
        mx = jnp.max(masked, axis=reduce_axes)
        acc = jnp.where(out_pos == j, mx[:, None], acc)
    return acc


def _topk_lanes(keys):
    return _successive_maxima(keys, (1,), keys.shape[0])


def _topk_slots(keys):
    return _successive_maxima(keys, (0, 2), keys.shape[1])


def kernel(x):
    m, n = x.shape
    slice_w = n // N_PARTS

    def body(x_ref, o_ref, cand_ref, yrecv_ref, send_sems, recv_sems):
        my_x = lax.axis_index("x")
        my_y = lax.axis_index("y")
        my_z = lax.axis_index("z")

        partners = [
            (my_x, my_y, my_z ^ 1),
            (my_x, my_y, my_z ^ 2),
            (1 - my_x, my_y, my_z),
            (my_x, 1 - my_y, my_z),
        ]

        barrier = pltpu.get_barrier_semaphore()
        for nbr in partners:
            pl.semaphore_signal(
                barrier, inc=1, device_id=nbr,
                device_id_type=pl.DeviceIdType.MESH,
            )
        pl.semaphore_wait(barrier, len(partners))

        p = my_x * 4 + my_z
        start = p * slice_w
        cols = (lax.broadcasted_iota(jnp.int32, (m, slice_w), 1) + start) & IDX_MASK
        keys = _to_keys(x_ref[:, pl.ds(start, slice_w)], cols)
        cand_ref[0] = _topk_lanes(keys)

        for s in range(3):
            w = 1 << s
            rdma = pltpu.make_async_remote_copy(
                src_ref=cand_ref.at[0:w],
                dst_ref=cand_ref.at[w:2 * w],
                send_sem=send_sems.at[s],
                recv_sem=recv_sems.at[s],
                device_id=partners[s],
                device_id_type=pl.DeviceIdType.MESH,
            )
            rdma.start()
            rdma.wait()

        acc = _topk_slots(cand_ref[...])

        cand_ref[0] = acc
        rdma = pltpu.make_async_remote_copy(
            src_ref=cand_ref.at[0],
            dst_ref=yrecv_ref,
            send_sem=send_sems.at[3],
            recv_sem=recv_sems.at[3],
            device_id=partners[3],
            device_id_type=pl.DeviceIdType.MESH,
        )
        rdma.start()
        rdma.wait()

        merged = jnp.concatenate([acc, yrecv_ref[...]], axis=1)
        pos = lax.broadcasted_iota(jnp.int32, (m, 2 * K), 1)
        merged = (merged & KEY_MASK) | pos
        o_ref[...] = _from_key(_topk_lanes(merged))

    return pl.pallas_call(
        body,
        out_shape=jax.ShapeDtypeStruct((m, K), jnp.float32),
        in_specs=[pl.BlockSpec(memory_space=pltpu.VMEM)],
        out_specs=pl.BlockSpec(memory_space=pltpu.VMEM),
        scratch_shapes=[
            pltpu.VMEM((N_PARTS, m, K), jnp.int32),
            pltpu.VMEM((m, K), jnp.int32),
            pltpu.SemaphoreType.DMA((4,)),
            pltpu.SemaphoreType.DMA((4,)),
        ],
        compiler_params=pltpu.CompilerParams(collective_id=0),
    )(x)


# device time: 21296 ns/iter; 2.6633x vs baseline; 2.6633x over previous
import jax
import jax.numpy as jnp
from jax import lax
from jax.experimental import pallas as pl
from jax.experimental.pallas import tpu as pltpu

K = 16
N_PARTS = 8
IDX_MASK = 0xFFF
KEY_MASK = ~0xFFF
NEG = -(2 ** 31)
SIGN_FIX = 0x7FFFFFFF


def _to_keys(vals_f32, col_iota):
    bits = lax.bitcast_convert_type(vals_f32, jnp.int32)
    mkey = jnp.where(bits >= 0, bits, bits ^ SIGN_FIX)
    return (mkey & KEY_MASK) | col_iota


def _from_key(key):
    mk = key & KEY_MASK
    bits = jnp.where(mk >= 0, mk, mk ^ SIGN_FIX)
    return lax.bitcast_convert_type(bits, jnp.float32)


def _topk(keys):
    rows = keys.shape[0]
    out_pos = lax.broadcasted_iota(jnp.int32, (rows, K), 1)
    mx = jnp.max(keys, axis=1, keepdims=True)
    acc = jnp.broadcast_to(mx, (rows, K))
    for j in range(1, K):
        mx = jnp.max(jnp.where(keys < mx, keys, NEG), axis=1, keepdims=True)
        acc = jnp.where(out_pos == j, mx, acc)
    return acc


def kernel(x):
    m, n = x.shape
    rows_per = m // N_PARTS

    def body(x_ref, o_ref, cand_ref, ysend_ref, yrecv_ref,
             ysend_sem, yrecv_sem, send_sems, recv_sems):
        my_x = lax.axis_index("x")
        my_y = lax.axis_index("y")
        my_z = lax.axis_index("z")
        p = my_x * 4 + my_z
        y_partner = (my_x, 1 - my_y, my_z)

        barrier = pltpu.get_barrier_semaphore()
        for q in range(N_PARTS):
            pl.semaphore_signal(
                barrier, inc=1, device_id=(q // 4, my_y, q % 4),
                device_id_type=pl.DeviceIdType.MESH,
            )
        pl.semaphore_signal(
            barrier, inc=1, device_id=y_partner,
            device_id_type=pl.DeviceIdType.MESH,
        )
        pl.semaphore_wait(barrier, N_PARTS + 1)

        block = x_ref[pl.ds(p * rows_per, rows_per), :]
        cols = lax.broadcasted_iota(jnp.int32, (rows_per, n), 1) & IDX_MASK
        acc = _topk(_to_keys(block, cols))

        ysend_ref[...] = acc
        rdma_y = pltpu.make_async_remote_copy(
            src_ref=ysend_ref,
            dst_ref=yrecv_ref,
            send_sem=ysend_sem,
            recv_sem=yrecv_sem,
            device_id=y_partner,
            device_id_type=pl.DeviceIdType.MESH,
        )
        rdma_y.start()
        rdma_y.wait()

        merged = jnp.concatenate([acc, yrecv_ref[...]], axis=1)
        pos = lax.broadcasted_iota(jnp.int32, (rows_per, 2 * K), 1)
        merged = (merged & KEY_MASK) | pos
        cand_ref[pl.ds(p * rows_per, rows_per), :] = _topk(merged)

        for q in range(N_PARTS):
            @pl.when(q != p)
            def _(q=q):
                send = pltpu.make_async_remote_copy(
                    src_ref=cand_ref.at[pl.ds(p * rows_per, rows_per), :],
                    dst_ref=cand_ref.at[pl.ds(p * rows_per, rows_per), :],
                    send_sem=send_sems.at[q],
                    recv_sem=recv_sems.at[p],
                    device_id=(q // 4, my_y, q % 4),
                    device_id_type=pl.DeviceIdType.MESH,
                )
                send.start()

        for q in range(N_PARTS):
            @pl.when(q != p)
            def _(q=q):
                done = pltpu.make_async_remote_copy(
                    src_ref=cand_ref.at[pl.ds(p * rows_per, rows_per), :],
                    dst_ref=cand_ref.at[pl.ds(q * rows_per, rows_per), :],
                    send_sem=send_sems.at[q],
                    recv_sem=recv_sems.at[q],
                    device_id=(q // 4, my_y, q % 4),
                    device_id_type=pl.DeviceIdType.MESH,
                )
                done.wait()

        o_ref[...] = _from_key(cand_ref[...])

    return pl.pallas_call(
        body,
        out_shape=jax.ShapeDtypeStruct((m, K), jnp.float32),
        in_specs=[pl.BlockSpec(memory_space=pltpu.VMEM)],
        out_specs=pl.BlockSpec(memory_space=pltpu.VMEM),
        scratch_shapes=[
            pltpu.VMEM((m, K), jnp.int32),
            pltpu.VMEM((m // N_PARTS, K), jnp.int32),
            pltpu.VMEM((m // N_PARTS, K), jnp.int32),
            pltpu.SemaphoreType.DMA,
            pltpu.SemaphoreType.DMA,
            pltpu.SemaphoreType.DMA((N_PARTS,)),
            pltpu.SemaphoreType.DMA((N_PARTS,)),
        ],
        compiler_params=pltpu.CompilerParams(collective_id=0),
    )(x)


# device time: 21206 ns/iter; 2.6746x vs baseline; 1.0042x over previous
import jax
import jax.numpy as jnp
from jax import lax
from jax.experimental import pallas as pl
from jax.experimental.pallas import tpu as pltpu

K = 16
N_PARTS = 8
IDX_MASK = 0xFFF
KEY_MASK = ~0xFFF
NEG = -(2 ** 31)
SIGN_FIX = 0x7FFFFFFF


def _to_keys(vals_f32, col_iota):
    bits = lax.bitcast_convert_type(vals_f32, jnp.int32)
    mkey = jnp.where(bits >= 0, bits, bits ^ SIGN_FIX)
    return (mkey & KEY_MASK) | col_iota


def _from_key(key):
    mk = key & KEY_MASK
    bits = jnp.where(mk >= 0, mk, mk ^ SIGN_FIX)
    return lax.bitcast_convert_type(bits, jnp.float32)


def _topk(keys):
    rows = keys.shape[0]
    out_pos = lax.broadcasted_iota(jnp.int32, (rows, K), 1)
    mx = jnp.max(keys, axis=1, keepdims=True)
    acc = jnp.broadcast_to(mx, (rows, K))
    for j in range(1, K):
        mx = jnp.max(jnp.where(keys < mx, keys, NEG), axis=1, keepdims=True)
        acc = jnp.where(out_pos == j, mx, acc)
    return acc


def kernel(x):
    m, n = x.shape
    rows_per = m // N_PARTS

    def body(x_ref, o_ref, cand_ref, send_sems, recv_sems):
        my_x = lax.axis_index("x")
        my_y = lax.axis_index("y")
        my_z = lax.axis_index("z")
        p = my_x * 4 + my_z

        def coords(t, q):
            return (q // 4, my_y if t == 0 else 1 - my_y, q % 4)

        barrier = pltpu.get_barrier_semaphore()
        for t in (0, 1):
            for q in range(N_PARTS):
                pl.semaphore_signal(
                    barrier, inc=1, device_id=coords(t, q),
                    device_id_type=pl.DeviceIdType.MESH,
                )

        block = x_ref[pl.ds(p * rows_per, rows_per), :]
        cols = lax.broadcasted_iota(jnp.int32, (rows_per, n), 1)
        cand_ref[0, pl.ds(p * rows_per, rows_per), :] = _topk(
            _to_keys(block, cols)
        )

        pl.semaphore_wait(barrier, 2 * N_PARTS)

        my_rows = pl.ds(p * rows_per, rows_per)
        for t in (0, 1):
            for q in range(N_PARTS):
                if t == 0:
                    send = pltpu.make_async_remote_copy(
                        src_ref=cand_ref.at[0, my_rows, :],
                        dst_ref=cand_ref.at[0, my_rows, :],
                        send_sem=send_sems.at[q],
                        recv_sem=recv_sems.at[p],
                        device_id=coords(0, q),
                        device_id_type=pl.DeviceIdType.MESH,
                    )
                    pl.when(q != p)(send.start)
                else:
                    send = pltpu.make_async_remote_copy(
                        src_ref=cand_ref.at[0, my_rows, :],
                        dst_ref=cand_ref.at[1, my_rows, :],
                        send_sem=send_sems.at[N_PARTS + q],
                        recv_sem=recv_sems.at[N_PARTS + p],
                        device_id=coords(1, q),
                        device_id_type=pl.DeviceIdType.MESH,
                    )
                    send.start()

        for t in (0, 1):
            for q in range(N_PARTS):
                done = pltpu.make_async_remote_copy(
                    src_ref=cand_ref.at[0, my_rows, :],
                    dst_ref=cand_ref.at[t, pl.ds(q * rows_per, rows_per), :],
                    send_sem=send_sems.at[t * N_PARTS + q],
                    recv_sem=recv_sems.at[t * N_PARTS + q],
                    device_id=coords(t, q),
                    device_id_type=pl.DeviceIdType.MESH,
                )
                if t == 0:
                    pl.when(q != p)(done.wait)
                else:
                    done.wait()

        merged = jnp.concatenate([cand_ref[0], cand_ref[1]], axis=1)
        pos = lax.broadcasted_iota(jnp.int32, (m, 2 * K), 1)
        merged = (merged & KEY_MASK) | pos
        o_ref[...] = _from_key(_topk(merged))

    return pl.pallas_call(
        body,
        out_shape=jax.ShapeDtypeStruct((m, K), jnp.float32),
        in_specs=[pl.BlockSpec(memory_space=pltpu.VMEM)],
        out_specs=pl.BlockSpec(memory_space=pltpu.VMEM),
        scratch_shapes=[
            pltpu.VMEM((2, m, K), jnp.int32),
            pltpu.SemaphoreType.DMA((2 * N_PARTS,)),
            pltpu.SemaphoreType.DMA((2 * N_PARTS,)),
        ],
        compiler_params=pltpu.CompilerParams(collective_id=0),
    )(x)
